# baseline (speedup 1.0000x reference)
import functools

import jax
import jax.numpy as jnp
from jax import lax
from jax.experimental import pallas as pl
from jax.experimental.pallas import tpu as pltpu
from jax.experimental.pallas import tpu_sc as plsc

B = 8
NPTS = 2048
D_OUT = 32
NNB = 15
ROWS = 1024

NC = 2
NS = 16
NW = NC * NS
CPB = NW // B
PW = NPTS // CPB
GRP = 16


def _tc_body(x0r_ref, x0t_ref, ft_ref, ws_ref, wn_ref, b2_ref,
             idx_ref, at_ref, ct_ref):
    x = x0r_ref[...]
    xt = x0t_ref[...]
    rr = jnp.sum(x * x, axis=1, keepdims=True)
    cc = jnp.sum(xt * xt, axis=0, keepdims=True)
    xy = jnp.dot(x, xt, preferred_element_type=jnp.float32)
    dist = jnp.sqrt(jnp.maximum(rr + cc - 2.0 * xy, 0.0))

    col = lax.broadcasted_iota(jnp.int32, (ROWS, NPTS), 1)
    inf = jnp.float32(jnp.inf)
    slot = lax.broadcasted_iota(jnp.int32, (ROWS, 16), 1)

    def pick(k, carry):
        d, acc = carry
        m = jnp.min(d, axis=1, keepdims=True)
        jm = jnp.min(jnp.where(d == m, col, NPTS), axis=1, keepdims=True)
        acc = jnp.where(slot == k - 1, jm, acc)
        d = jnp.where(col == jm, inf, d)
        return d, acc

    acc0 = jnp.zeros((ROWS, 16), jnp.int32)
    _, acc = lax.fori_loop(0, NNB + 1, pick, (dist, acc0), unroll=True)
    idx_ref[...] = acc

    ft = ft_ref[...]
    at_ref[...] = jnp.dot(ws_ref[...], ft, preferred_element_type=jnp.float32,
                          precision=lax.Precision.HIGHEST) + b2_ref[...]
    ct_ref[...] = jnp.dot(wn_ref[...], ft, preferred_element_type=jnp.float32,
                          precision=lax.Precision.HIGHEST)


_tc_call = pl.pallas_call(
    _tc_body,
    grid=(B, NPTS // ROWS),
    in_specs=[
        pl.BlockSpec((None, ROWS, 3), lambda bb, rr: (bb, rr, 0)),
        pl.BlockSpec((None, 3, NPTS), lambda bb, rr: (bb, 0, 0)),
        pl.BlockSpec((None, 6, ROWS), lambda bb, rr: (bb, 0, rr)),
        pl.BlockSpec((D_OUT, 6), lambda bb, rr: (0, 0)),
        pl.BlockSpec((D_OUT, 6), lambda bb, rr: (0, 0)),
        pl.BlockSpec((D_OUT, 1), lambda bb, rr: (0, 0)),
    ],
    out_specs=[
        pl.BlockSpec((None, ROWS, 16), lambda bb, rr: (bb, rr, 0)),
        pl.BlockSpec((None, D_OUT, ROWS), lambda bb, rr: (bb, 0, rr)),
        pl.BlockSpec((None, D_OUT, ROWS), lambda bb, rr: (bb, 0, rr)),
    ],
    out_shape=[
        jax.ShapeDtypeStruct((B, NPTS, 16), jnp.int32),
        jax.ShapeDtypeStruct((B, D_OUT, NPTS), jnp.float32),
        jax.ShapeDtypeStruct((B, D_OUT, NPTS), jnp.float32),
    ],
    compiler_params=pltpu.CompilerParams(
        dimension_semantics=("parallel", "parallel")),
)


@functools.partial(
    pl.kernel,
    out_type=jax.ShapeDtypeStruct((B, NPTS * D_OUT), jnp.float32),
    mesh=plsc.VectorSubcoreMesh(core_axis_name="c", subcore_axis_name="s"),
    compiler_params=pltpu.CompilerParams(needs_layout_passes=False),
    scratch_types=[
        pltpu.VMEM((D_OUT * NPTS,), jnp.float32),
        pltpu.VMEM((D_OUT, PW), jnp.float32),
        pltpu.VMEM((PW * 16,), jnp.int32),
        pltpu.VMEM((PW * D_OUT,), jnp.float32),
    ],
)
def _sc_call(at_hbm, ct_hbm, idx_hbm, out_hbm, ct_v, at_v, idx_v, out_v):
    wid = lax.axis_index("s") * NC + lax.axis_index("c")
    bb = wid // CPB
    p0 = (wid % CPB) * PW
    pltpu.sync_copy(ct_hbm.at[bb], ct_v)
    pltpu.sync_copy(at_hbm.at[bb, :, pl.ds(p0, PW)], at_v)
    pltpu.sync_copy(idx_hbm.at[bb, pl.ds(p0 * 16, PW * 16)], idx_v)

    lanes = lax.iota(jnp.int32, 16)
    inv = jnp.float32(1.0 / NNB)

    def group(g, carry):
        p = g * GRP
        idx_base = p * 16 + lanes * 16
        out_base = p * D_OUT + lanes * D_OUT
        jvs = [plsc.load_gather(idx_v, [idx_base + k]) for k in range(NNB)]
        for d in range(D_OUT):
            a = at_v[d, pl.ds(p, GRP)]
            acc = jnp.zeros((16,), jnp.float32)
            for k in range(NNB):
                c = plsc.load_gather(ct_v, [jvs[k] + d * NPTS])
                acc = acc + jnp.maximum(a + c, 0.0)
            plsc.store_scatter(out_v, [out_base + d], acc * inv)
        return carry

    lax.fori_loop(0, PW // GRP, group, 0)
    pltpu.sync_copy(out_v, out_hbm.at[bb, pl.ds(p0 * D_OUT, PW * D_OUT)])


def kernel(x0, v0, W, b):
    feat = jnp.concatenate([x0, v0], axis=-1)
    ft = jnp.transpose(feat, (0, 2, 1))
    x0t = jnp.transpose(x0, (0, 2, 1))
    ws = jnp.concatenate([W[:, 0:3], W[:, 6:9]], axis=1)
    wn = jnp.concatenate([W[:, 3:6], W[:, 9:12]], axis=1)
    b2 = b[:, None]
    idxs, a_t, c_t = _tc_call(x0, x0t, ft, ws, wn, b2)
    out = _sc_call(a_t, c_t.reshape(B, D_OUT * NPTS),
                   idxs.reshape(B, NPTS * 16))
    return out, v0


# speedup vs baseline: 1.1344x; 1.1344x over previous
import functools

import jax
import jax.numpy as jnp
from jax import lax
from jax.experimental import pallas as pl
from jax.experimental.pallas import tpu as pltpu
from jax.experimental.pallas import tpu_sc as plsc

B = 8
NPTS = 2048
D_OUT = 32
NNB = 15
ROWS = 1024

NC = 2
NS = 16
NW = NC * NS
CPB = NW // B
PW = NPTS // CPB
GRP = 16


def _tc_body(x0r_ref, x0t_ref, ft_ref, ws_ref, wn_ref, b2_ref,
             idx_ref, at_ref, ct_ref):
    x = x0r_ref[...]
    xt = x0t_ref[...]
    rr = jnp.sum(x * x, axis=1, keepdims=True)
    cc = jnp.sum(xt * xt, axis=0, keepdims=True)
    xy = jnp.dot(x, xt, preferred_element_type=jnp.float32)
    dist = jnp.sqrt(jnp.maximum(rr + cc - 2.0 * xy, 0.0))

    col = lax.broadcasted_iota(jnp.int32, (ROWS, NPTS), 1)
    inf = jnp.float32(jnp.inf)
    slot = lax.broadcasted_iota(jnp.int32, (ROWS, 16), 1)

    def pick(k, carry):
        d, acc = carry
        m = jnp.min(d, axis=1, keepdims=True)
        jm = jnp.min(jnp.where(d == m, col, NPTS), axis=1, keepdims=True)
        acc = jnp.where(slot == k - 1, jm, acc)
        d = jnp.where(col == jm, inf, d)
        return d, acc

    acc0 = jnp.zeros((ROWS, 16), jnp.int32)
    _, acc = lax.fori_loop(0, NNB + 1, pick, (dist, acc0), unroll=8)
    idx_ref[...] = acc

    ft = ft_ref[...]
    at_ref[...] = jnp.dot(ws_ref[...], ft, preferred_element_type=jnp.float32,
                          precision=lax.Precision.HIGHEST) + b2_ref[...]
    ct_ref[...] = jnp.dot(wn_ref[...], ft, preferred_element_type=jnp.float32,
                          precision=lax.Precision.HIGHEST)


_tc_call = pl.pallas_call(
    _tc_body,
    grid=(B, NPTS // ROWS),
    in_specs=[
        pl.BlockSpec((None, ROWS, 3), lambda bb, rr: (bb, rr, 0)),
        pl.BlockSpec((None, 3, NPTS), lambda bb, rr: (bb, 0, 0)),
        pl.BlockSpec((None, 6, ROWS), lambda bb, rr: (bb, 0, rr)),
        pl.BlockSpec((D_OUT, 6), lambda bb, rr: (0, 0)),
        pl.BlockSpec((D_OUT, 6), lambda bb, rr: (0, 0)),
        pl.BlockSpec((D_OUT, 1), lambda bb, rr: (0, 0)),
    ],
    out_specs=[
        pl.BlockSpec((None, ROWS, 16), lambda bb, rr: (bb, rr, 0)),
        pl.BlockSpec((None, D_OUT, ROWS), lambda bb, rr: (bb, 0, rr)),
        pl.BlockSpec((None, D_OUT, ROWS), lambda bb, rr: (bb, 0, rr)),
    ],
    out_shape=[
        jax.ShapeDtypeStruct((B, NPTS, 16), jnp.int32),
        jax.ShapeDtypeStruct((B, D_OUT, NPTS), jnp.float32),
        jax.ShapeDtypeStruct((B, D_OUT, NPTS), jnp.float32),
    ],
    compiler_params=pltpu.CompilerParams(
        dimension_semantics=("parallel", "parallel")),
)


@functools.partial(
    pl.kernel,
    out_type=jax.ShapeDtypeStruct((B, NPTS * D_OUT), jnp.float32),
    mesh=plsc.VectorSubcoreMesh(core_axis_name="c", subcore_axis_name="s"),
    compiler_params=pltpu.CompilerParams(needs_layout_passes=False),
    scratch_types=[
        pltpu.VMEM((D_OUT * NPTS,), jnp.float32),
        pltpu.VMEM((D_OUT, PW), jnp.float32),
        pltpu.VMEM((PW * 16,), jnp.int32),
        pltpu.VMEM((PW * D_OUT,), jnp.float32),
    ],
)
def _sc_call(at_hbm, ct_hbm, idx_hbm, out_hbm, ct_v, at_v, idx_v, out_v):
    wid = lax.axis_index("s") * NC + lax.axis_index("c")
    bb = wid // CPB
    p0 = (wid % CPB) * PW
    pltpu.sync_copy(ct_hbm.at[bb], ct_v)
    pltpu.sync_copy(at_hbm.at[bb, :, pl.ds(p0, PW)], at_v)
    pltpu.sync_copy(idx_hbm.at[bb, pl.ds(p0 * 16, PW * 16)], idx_v)

    lanes = lax.iota(jnp.int32, 16)
    inv = jnp.float32(1.0 / NNB)

    def group(g, carry):
        p = g * GRP
        idx_base = p * 16 + lanes * 16
        out_base = p * D_OUT + lanes * D_OUT
        jvs = [plsc.load_gather(idx_v, [idx_base + k]) for k in range(NNB)]
        for d in range(D_OUT):
            a = at_v[d, pl.ds(p, GRP)]
            acc = jnp.zeros((16,), jnp.float32)
            for k in range(NNB):
                c = plsc.load_gather(ct_v, [jvs[k] + d * NPTS])
                acc = acc + jnp.maximum(a + c, 0.0)
            plsc.store_scatter(out_v, [out_base + d], acc * inv)
        return carry

    lax.fori_loop(0, PW // GRP, group, 0)
    pltpu.sync_copy(out_v, out_hbm.at[bb, pl.ds(p0 * D_OUT, PW * D_OUT)])


def kernel(x0, v0, W, b):
    feat = jnp.concatenate([x0, v0], axis=-1)
    ft = jnp.transpose(feat, (0, 2, 1))
    x0t = jnp.transpose(x0, (0, 2, 1))
    ws = jnp.concatenate([W[:, 0:3], W[:, 6:9]], axis=1)
    wn = jnp.concatenate([W[:, 3:6], W[:, 9:12]], axis=1)
    b2 = b[:, None]
    idxs, a_t, c_t = _tc_call(x0, x0t, ft, ws, wn, b2)
    out = _sc_call(a_t, c_t.reshape(B, D_OUT * NPTS),
                   idxs.reshape(B, NPTS * 16))
    return out, v0
